# baseline (speedup 1.0000x reference)
import functools

import jax
import jax.numpy as jnp
from jax import lax
from jax.experimental import pallas as pl
from jax.experimental.pallas import tpu as pltpu
from jax.experimental.pallas import tpu_sc as plsc

N_NODES = 10000
NPAD = 10240
N_EDGES = 320000
NCORES = 2
NSUB = 16
LANES = 16
HALF = 64
K = 80
EPS = N_EDGES // NSUB
NCHUNK = EPS // K
ROWS_PER_SUB = NPAD // NSUB


NSLOT = 4


def _sc_accumulate_body(with_cnt, xcat, src, dst, u, s_out, t_out, c_out,
                        s_sh, t_sh, c_sh, sidx, didx0, didx1, didx2, didx3,
                        ubuf, gbuf, sbuf, tbuf, ones, zc, *sems):
  didxs = (didx0, didx1, didx2, didx3)
  idx_sems = sems[0:NSLOT]
  g_sems = sems[NSLOT:2 * NSLOT]
  scs_sems = sems[2 * NSLOT:3 * NSLOT]
  sct_sems = sems[3 * NSLOT:4 * NSLOT]
  c = lax.axis_index("c")
  s = lax.axis_index("s")

  zero16 = jnp.zeros((LANES,), jnp.float32)
  one16 = jnp.ones((LANES,), jnp.float32)
  for r in range(K):
    for j in range(HALF // LANES):
      sbuf[0, r, pl.ds(j * LANES, LANES)] = zero16
    zc[r, :] = zero16
    ones[r, :] = one16

  row0 = s * ROWS_PER_SUB
  for b in range(ROWS_PER_SUB // K):
    sl = pl.ds(row0 + b * K, K)
    pltpu.sync_copy(sbuf.at[0], s_sh.at[sl])
    pltpu.sync_copy(sbuf.at[0], t_sh.at[sl])
    if with_cnt:
      pltpu.sync_copy(zc, c_sh.at[sl])
  plsc.subcore_barrier()

  def issue_idx(g, b):
    base = s * EPS + g * K
    pltpu.async_copy(src.at[pl.ds(base, K)], sidx.at[b], idx_sems[b])
    pltpu.async_copy(dst.at[pl.ds(base, K)], didxs[b], idx_sems[b])
    pltpu.async_copy(u.at[pl.ds(base, K)], ubuf.at[b], idx_sems[b])

  def wait_idx(b):
    pltpu.make_async_copy(src.at[pl.ds(0, K)], sidx.at[b], idx_sems[b]).wait()
    pltpu.make_async_copy(dst.at[pl.ds(0, K)], didxs[b], idx_sems[b]).wait()
    pltpu.make_async_copy(u.at[pl.ds(0, K)], ubuf.at[b], idx_sems[b]).wait()

  def fixup(b):
    for j in range(K // LANES):
      sl = pl.ds(j * LANES, LANES)
      sidx[b, sl] = sidx[b, sl] * 2 + c

  def issue_gather(b):
    pltpu.async_copy(xcat.at[sidx.at[b]], gbuf.at[b], g_sems[b])

  def wait_gather(b):
    pltpu.make_async_copy(xcat.at[sidx.at[b]], gbuf.at[b], g_sems[b]).wait()

  def issue_scs(b):
    pltpu.async_copy(sbuf.at[b % 2], s_sh.at[didxs[b]], scs_sems[b], add=True)
    if with_cnt:
      @pl.when(c == 0)
      def _():
        pltpu.async_copy(ones, c_sh.at[didxs[b]], scs_sems[b], add=True)

  def wait_scs(b):
    pltpu.make_async_copy(sbuf.at[b % 2], s_sh.at[didxs[b]], scs_sems[b]).wait()
    if with_cnt:
      @pl.when(c == 0)
      def _():
        pltpu.make_async_copy(ones, c_sh.at[didxs[b]], scs_sems[b]).wait()

  def multiply(b):
    tb = b % 2
    def mul16(i16, carry):
      base_i = i16 * LANES
      uv = ubuf[b, pl.ds(base_i, LANES)]
      for di in range(LANES):
        i = base_i + di
        u16 = jnp.take_along_axis(uv, jnp.full((LANES,), di, jnp.int32),
                                  axis=0)
        for j2 in range(HALF // (2 * LANES)):
          v = gbuf[b, i, pl.ds(j2 * 2 * LANES, 2 * LANES)]
          lo, hi = plsc.unpack(v, format=plsc.PackFormat.INTERLEAVED,
                               preferred_element_type=jnp.float32)
          sl_lo = pl.ds(j2 * 2 * LANES, LANES)
          sl_hi = pl.ds(j2 * 2 * LANES + LANES, LANES)
          sbuf[tb, i, sl_lo] = lo
          sbuf[tb, i, sl_hi] = hi
          tbuf[tb, i, sl_lo] = lo * u16
          tbuf[tb, i, sl_hi] = hi * u16
      return carry
    lax.fori_loop(0, K // LANES, mul16, 0)

  def issue_sct(b):
    pltpu.async_copy(tbuf.at[b % 2], t_sh.at[didxs[b]], sct_sems[b], add=True)

  def wait_sct(b):
    pltpu.make_async_copy(tbuf.at[b % 2], t_sh.at[didxs[b]], sct_sems[b]).wait()

  issue_idx(0, 0)
  issue_idx(1, 1)
  wait_idx(0)
  fixup(0)
  issue_gather(0)

  def slot(g, b, drain=True, idx2=True, nxt=True):
    b1 = (b + 1) % NSLOT
    b2 = (b + 2) % NSLOT
    wait_gather(b)
    if nxt:
      wait_idx(b1)
      fixup(b1)
      issue_gather(b1)
    if drain:
      wait_sct(b2)
      wait_scs(b2)
    if idx2:
      issue_idx(g + 2, b2)
    multiply(b)
    issue_scs(b)
    issue_sct(b)

  def outer_body(oi, carry):
    for b in range(NSLOT):
      g = oi * NSLOT + b
      b1 = (b + 1) % NSLOT
      b2 = (b + 2) % NSLOT
      wait_gather(b)
      wait_idx(b1)
      fixup(b1)
      issue_gather(b1)

      @pl.when(g >= 2)
      def _():
        wait_sct(b2)
        wait_scs(b2)
      issue_idx(g + 2, b2)
      multiply(b)
      issue_scs(b)
      issue_sct(b)
    return carry

  assert (NCHUNK - 2) % NSLOT == 0
  lax.fori_loop(0, (NCHUNK - 2) // NSLOT, outer_body, 0)

  slot(NCHUNK - 2, (NCHUNK - 2) % NSLOT, idx2=False)
  slot(NCHUNK - 1, (NCHUNK - 1) % NSLOT, idx2=False, nxt=False)
  wait_sct((NCHUNK - 2) % NSLOT)
  wait_scs((NCHUNK - 2) % NSLOT)
  wait_sct((NCHUNK - 1) % NSLOT)
  wait_scs((NCHUNK - 1) % NSLOT)
  plsc.subcore_barrier()

  for b in range(ROWS_PER_SUB // K):
    r = row0 + b * K
    sl = pl.ds(r, K)
    csl = pl.ds(c * HALF, HALF)
    pltpu.sync_copy(s_sh.at[sl], s_out.at[pl.ds(r, K), csl])
    pltpu.sync_copy(t_sh.at[sl], t_out.at[pl.ds(r, K), csl])
    if with_cnt:
      @pl.when(c == 0)
      def _():
        pltpu.sync_copy(c_sh.at[sl], c_out.at[sl])


def _make_sc_accumulate(with_cnt):
  out_type = [
      jax.ShapeDtypeStruct((NPAD, NCORES * HALF), jnp.float32),
      jax.ShapeDtypeStruct((NPAD, NCORES * HALF), jnp.float32),
  ]
  if with_cnt:
    out_type.append(jax.ShapeDtypeStruct((NPAD, LANES), jnp.float32))
  scratch = [
      pltpu.VMEM_SHARED((NPAD, HALF), jnp.float32),
      pltpu.VMEM_SHARED((NPAD, HALF), jnp.float32),
      pltpu.VMEM_SHARED((NPAD, LANES), jnp.float32),
      pltpu.VMEM((NSLOT, K), jnp.int32),
      pltpu.VMEM((K,), jnp.int32),
      pltpu.VMEM((K,), jnp.int32),
      pltpu.VMEM((K,), jnp.int32),
      pltpu.VMEM((K,), jnp.int32),
      pltpu.VMEM((NSLOT, K), jnp.float32),
      pltpu.VMEM((NSLOT, K, HALF), jnp.bfloat16),
      pltpu.VMEM((2, K, HALF), jnp.float32),
      pltpu.VMEM((2, K, HALF), jnp.float32),
      pltpu.VMEM((K, LANES), jnp.float32),
      pltpu.VMEM((K, LANES), jnp.float32),
  ] + [pltpu.SemaphoreType.DMA] * (4 * NSLOT)
  mesh = plsc.VectorSubcoreMesh(
      core_axis_name="c", subcore_axis_name="s",
      num_cores=NCORES, num_subcores=NSUB)
  body = functools.partial(_sc_accumulate_body, with_cnt)
  if not with_cnt:
    def body(xcat, src, dst, u, s_out, t_out, *rest):
      return _sc_accumulate_body(False, xcat, src, dst, u, s_out, t_out,
                                 None, *rest)
  return pl.kernel(body, out_type=tuple(out_type), mesh=mesh,
                   scratch_types=tuple(scratch),
                   compiler_params=pltpu.CompilerParams(
                       use_tc_tiling_on_sc=False,
                       needs_layout_passes=False))


def _dense_body(last, S_ref, T_ref, X_ref, cnt_ref, W0_ref, W1_ref, Wr_ref,
                b_ref, o_ref):
  Sb = S_ref[...]
  Tb = T_ref[...]
  inv = 1.0 / jnp.clip(cnt_ref[...], 1.0, None)
  z = (jnp.dot(Sb - Tb, W0_ref[...], preferred_element_type=jnp.float32)
       + jnp.dot(Tb, W1_ref[...], preferred_element_type=jnp.float32))
  z = z * inv
  z = z + jnp.dot(X_ref[...], Wr_ref[...], preferred_element_type=jnp.float32)
  z = z + b_ref[...]
  z = jnp.maximum(z, 0.0)
  if last:
    m = jnp.max(z, axis=1, keepdims=True)
    e = jnp.exp(z - m)
    z = z - m - jnp.log(jnp.sum(e, axis=1, keepdims=True))
  o_ref[...] = z


def _make_dense(d_in, d_out, last, br=1000):
  grid = (N_NODES // br,)
  return pl.pallas_call(
      functools.partial(_dense_body, last),
      grid=grid,
      in_specs=[
          pl.BlockSpec((br, d_in), lambda i: (i, 0)),
          pl.BlockSpec((br, d_in), lambda i: (i, 0)),
          pl.BlockSpec((br, d_in), lambda i: (i, 0)),
          pl.BlockSpec((br, 1), lambda i: (i, 0)),
          pl.BlockSpec((d_in, d_out), lambda i: (0, 0)),
          pl.BlockSpec((d_in, d_out), lambda i: (0, 0)),
          pl.BlockSpec((d_in, d_out), lambda i: (0, 0)),
          pl.BlockSpec((1, d_out), lambda i: (0, 0)),
      ],
      out_specs=pl.BlockSpec((br, d_out), lambda i: (i, 0)),
      out_shape=jax.ShapeDtypeStruct((N_NODES, d_out), jnp.float32),
  )


_make_sc_accumulate = functools.lru_cache(None)(_make_sc_accumulate)
_dense1 = _make_dense(128, 128, last=False)
_dense2 = _make_dense(128, 64, last=True)


def _split_cat(h):
  return h.astype(jnp.bfloat16).reshape(2 * N_NODES, HALF)


def _unpack_perm():
  p = []
  for blk in range(128 // (2 * LANES)):
    base = blk * 2 * LANES
    p.extend(base + 2 * i for i in range(LANES))
    p.extend(base + 2 * i + 1 for i in range(LANES))
  return jnp.array(p, dtype=jnp.int32)


def kernel(x, edge_index, edge_attr, W0_1, W1_1, Wr1, b1, W0_2, W1_2, Wr2, b2):
  src = edge_index[0]
  dst = edge_index[1]
  u = edge_attr[:, 0]

  perm = _unpack_perm()
  S2, T2, C = _make_sc_accumulate(True)(_split_cat(x), src, dst, u)
  S = S2[:N_NODES]
  T = T2[:N_NODES]
  cnt = C[:N_NODES, 0:1]
  h = _dense1(S, T, x, cnt, W0_1[perm], W1_1[perm], Wr1, b1[None, :])

  S2b, T2b = _make_sc_accumulate(False)(_split_cat(h), src, dst, u)
  Sb = S2b[:N_NODES]
  Tb = T2b[:N_NODES]
  return _dense2(Sb, Tb, h, cnt, W0_2[perm], W1_2[perm], Wr2, b2[None, :])


# speedup vs baseline: 1.3358x; 1.3358x over previous
import functools

import jax
import jax.numpy as jnp
from jax import lax
from jax.experimental import pallas as pl
from jax.experimental.pallas import tpu as pltpu
from jax.experimental.pallas import tpu_sc as plsc

N_NODES = 10000
NPAD = 10240
N_EDGES = 320000
NCORES = 2
NSUB = 16
LANES = 16
HALF = 64
K = 80
EPS = N_EDGES // NSUB
NCHUNK = EPS // K
ROWS_PER_SUB = NPAD // NSUB


NSLOT = 4


def _sc_accumulate_body(with_cnt, xcat, src, dst, u, s_out, t_out, c_out,
                        s_sh, t_sh, c_sh, sidx, didx0, didx1, didx2, didx3,
                        ubuf, gbuf, tbuf, ones, zc, *sems):
  didxs = (didx0, didx1, didx2, didx3)
  idx_sems = sems[0:NSLOT]
  g_sems = sems[NSLOT:2 * NSLOT]
  scs_sems = sems[2 * NSLOT:3 * NSLOT]
  sct_sems = sems[3 * NSLOT:4 * NSLOT]
  c = lax.axis_index("c")
  s = lax.axis_index("s")

  zero16 = jnp.zeros((LANES,), jnp.float32)
  one16 = jnp.ones((LANES,), jnp.float32)
  for r in range(K):
    for j in range(HALF // LANES):
      gbuf[0, r, pl.ds(j * LANES, LANES)] = zero16
    zc[r, :] = zero16
    ones[r, :] = one16

  row0 = s * ROWS_PER_SUB
  for b in range(ROWS_PER_SUB // K):
    sl = pl.ds(row0 + b * K, K)
    pltpu.sync_copy(gbuf.at[0], s_sh.at[sl])
    pltpu.sync_copy(gbuf.at[0], t_sh.at[sl])
    if with_cnt:
      pltpu.sync_copy(zc, c_sh.at[sl])
  plsc.subcore_barrier()

  def issue_idx(g, b):
    base = s * EPS + g * K
    pltpu.async_copy(src.at[pl.ds(base, K)], sidx.at[b], idx_sems[b])
    pltpu.async_copy(dst.at[pl.ds(base, K)], didxs[b], idx_sems[b])
    pltpu.async_copy(u.at[pl.ds(base, K)], ubuf.at[b], idx_sems[b])

  def wait_idx(b):
    pltpu.make_async_copy(src.at[pl.ds(0, K)], sidx.at[b], idx_sems[b]).wait()
    pltpu.make_async_copy(dst.at[pl.ds(0, K)], didxs[b], idx_sems[b]).wait()
    pltpu.make_async_copy(u.at[pl.ds(0, K)], ubuf.at[b], idx_sems[b]).wait()

  def fixup(b):
    for j in range(K // LANES):
      sl = pl.ds(j * LANES, LANES)
      sidx[b, sl] = sidx[b, sl] * 2 + c

  def issue_gather(b):
    pltpu.async_copy(xcat.at[sidx.at[b]], gbuf.at[b], g_sems[b])

  def wait_gather(b):
    pltpu.make_async_copy(xcat.at[sidx.at[b]], gbuf.at[b], g_sems[b]).wait()

  def issue_scs(b):
    pltpu.async_copy(gbuf.at[b], s_sh.at[didxs[b]], scs_sems[b], add=True)
    if with_cnt:
      @pl.when(c == 0)
      def _():
        pltpu.async_copy(ones, c_sh.at[didxs[b]], scs_sems[b], add=True)

  def wait_scs(b):
    pltpu.make_async_copy(gbuf.at[b], s_sh.at[didxs[b]], scs_sems[b]).wait()
    if with_cnt:
      @pl.when(c == 0)
      def _():
        pltpu.make_async_copy(ones, c_sh.at[didxs[b]], scs_sems[b]).wait()

  def multiply(b):
    tb = b % 2
    def mul16(i16, carry):
      base_i = i16 * LANES
      uv = ubuf[b, pl.ds(base_i, LANES)]
      for di in range(LANES):
        i = base_i + di
        u16 = jnp.take_along_axis(uv, jnp.full((LANES,), di, jnp.int32),
                                  axis=0)
        for j in range(HALF // LANES):
          sl = pl.ds(j * LANES, LANES)
          tbuf[tb, i, sl] = gbuf[b, i, sl] * u16
      return carry
    lax.fori_loop(0, K // LANES, mul16, 0)

  def issue_sct(b):
    pltpu.async_copy(tbuf.at[b % 2], t_sh.at[didxs[b]], sct_sems[b], add=True)

  def wait_sct(b):
    pltpu.make_async_copy(tbuf.at[b % 2], t_sh.at[didxs[b]], sct_sems[b]).wait()

  issue_idx(0, 0)
  issue_idx(1, 1)
  wait_idx(0)
  fixup(0)
  issue_gather(0)

  def slot(g, b, drain=True, idx2=True, nxt=True):
    b1 = (b + 1) % NSLOT
    b2 = (b + 2) % NSLOT
    wait_gather(b)
    if nxt:
      wait_idx(b1)
      fixup(b1)
      issue_gather(b1)
    issue_scs(b)
    if drain:
      wait_sct(b2)
      wait_scs(b2)
    if idx2:
      issue_idx(g + 2, b2)
    multiply(b)
    issue_sct(b)

  def outer_body(oi, carry):
    for b in range(NSLOT):
      g = oi * NSLOT + b
      b1 = (b + 1) % NSLOT
      b2 = (b + 2) % NSLOT
      wait_gather(b)
      wait_idx(b1)
      fixup(b1)
      issue_gather(b1)
      issue_scs(b)

      @pl.when(g >= 2)
      def _():
        wait_sct(b2)
        wait_scs(b2)
      issue_idx(g + 2, b2)
      multiply(b)
      issue_sct(b)
    return carry

  assert (NCHUNK - 2) % NSLOT == 0
  lax.fori_loop(0, (NCHUNK - 2) // NSLOT, outer_body, 0)

  slot(NCHUNK - 2, (NCHUNK - 2) % NSLOT, idx2=False)
  slot(NCHUNK - 1, (NCHUNK - 1) % NSLOT, idx2=False, nxt=False)
  wait_sct((NCHUNK - 2) % NSLOT)
  wait_scs((NCHUNK - 2) % NSLOT)
  wait_sct((NCHUNK - 1) % NSLOT)
  wait_scs((NCHUNK - 1) % NSLOT)
  plsc.subcore_barrier()

  for b in range(ROWS_PER_SUB // K):
    r = row0 + b * K
    sl = pl.ds(r, K)
    csl = pl.ds(c * HALF, HALF)
    pltpu.sync_copy(s_sh.at[sl], s_out.at[pl.ds(r, K), csl])
    pltpu.sync_copy(t_sh.at[sl], t_out.at[pl.ds(r, K), csl])
    if with_cnt:
      @pl.when(c == 0)
      def _():
        pltpu.sync_copy(c_sh.at[sl], c_out.at[sl])


def _make_sc_accumulate(with_cnt):
  out_type = [
      jax.ShapeDtypeStruct((NPAD, NCORES * HALF), jnp.float32),
      jax.ShapeDtypeStruct((NPAD, NCORES * HALF), jnp.float32),
  ]
  if with_cnt:
    out_type.append(jax.ShapeDtypeStruct((NPAD, LANES), jnp.float32))
  scratch = [
      pltpu.VMEM_SHARED((NPAD, HALF), jnp.float32),
      pltpu.VMEM_SHARED((NPAD, HALF), jnp.float32),
      pltpu.VMEM_SHARED((NPAD, LANES), jnp.float32),
      pltpu.VMEM((NSLOT, K), jnp.int32),
      pltpu.VMEM((K,), jnp.int32),
      pltpu.VMEM((K,), jnp.int32),
      pltpu.VMEM((K,), jnp.int32),
      pltpu.VMEM((K,), jnp.int32),
      pltpu.VMEM((NSLOT, K), jnp.float32),
      pltpu.VMEM((NSLOT, K, HALF), jnp.float32),
      pltpu.VMEM((2, K, HALF), jnp.float32),
      pltpu.VMEM((K, LANES), jnp.float32),
      pltpu.VMEM((K, LANES), jnp.float32),
  ] + [pltpu.SemaphoreType.DMA] * (4 * NSLOT)
  mesh = plsc.VectorSubcoreMesh(
      core_axis_name="c", subcore_axis_name="s",
      num_cores=NCORES, num_subcores=NSUB)
  body = functools.partial(_sc_accumulate_body, with_cnt)
  if not with_cnt:
    def body(xcat, src, dst, u, s_out, t_out, *rest):
      return _sc_accumulate_body(False, xcat, src, dst, u, s_out, t_out,
                                 None, *rest)
  return pl.kernel(body, out_type=tuple(out_type), mesh=mesh,
                   scratch_types=tuple(scratch),
                   compiler_params=pltpu.CompilerParams(
                       use_tc_tiling_on_sc=False))


def _dense_body(last, S_ref, T_ref, X_ref, cnt_ref, W0_ref, W1_ref, Wr_ref,
                b_ref, o_ref):
  Sb = S_ref[...]
  Tb = T_ref[...]
  inv = 1.0 / jnp.clip(cnt_ref[...], 1.0, None)
  z = (jnp.dot(Sb - Tb, W0_ref[...], preferred_element_type=jnp.float32)
       + jnp.dot(Tb, W1_ref[...], preferred_element_type=jnp.float32))
  z = z * inv
  z = z + jnp.dot(X_ref[...], Wr_ref[...], preferred_element_type=jnp.float32)
  z = z + b_ref[...]
  z = jnp.maximum(z, 0.0)
  if last:
    m = jnp.max(z, axis=1, keepdims=True)
    e = jnp.exp(z - m)
    z = z - m - jnp.log(jnp.sum(e, axis=1, keepdims=True))
  o_ref[...] = z


def _make_dense(d_in, d_out, last, br=1000):
  grid = (N_NODES // br,)
  return pl.pallas_call(
      functools.partial(_dense_body, last),
      grid=grid,
      in_specs=[
          pl.BlockSpec((br, d_in), lambda i: (i, 0)),
          pl.BlockSpec((br, d_in), lambda i: (i, 0)),
          pl.BlockSpec((br, d_in), lambda i: (i, 0)),
          pl.BlockSpec((br, 1), lambda i: (i, 0)),
          pl.BlockSpec((d_in, d_out), lambda i: (0, 0)),
          pl.BlockSpec((d_in, d_out), lambda i: (0, 0)),
          pl.BlockSpec((d_in, d_out), lambda i: (0, 0)),
          pl.BlockSpec((1, d_out), lambda i: (0, 0)),
      ],
      out_specs=pl.BlockSpec((br, d_out), lambda i: (i, 0)),
      out_shape=jax.ShapeDtypeStruct((N_NODES, d_out), jnp.float32),
  )


_make_sc_accumulate = functools.lru_cache(None)(_make_sc_accumulate)
_dense1 = _make_dense(128, 128, last=False)
_dense2 = _make_dense(128, 64, last=True)


def _split_cat(h):
  return h.reshape(2 * N_NODES, HALF)


def kernel(x, edge_index, edge_attr, W0_1, W1_1, Wr1, b1, W0_2, W1_2, Wr2, b2):
  src = edge_index[0]
  dst = edge_index[1]
  u = edge_attr[:, 0]

  S2, T2, C = _make_sc_accumulate(True)(_split_cat(x), src, dst, u)
  S = S2[:N_NODES]
  T = T2[:N_NODES]
  cnt = C[:N_NODES, 0:1]
  h = _dense1(S, T, x, cnt, W0_1, W1_1, Wr1, b1[None, :])

  S2b, T2b = _make_sc_accumulate(False)(_split_cat(h), src, dst, u)
  Sb = S2b[:N_NODES]
  Tb = T2b[:N_NODES]
  return _dense2(Sb, Tb, h, cnt, W0_2, W1_2, Wr2, b2[None, :])


# speedup vs baseline: 1.3367x; 1.0006x over previous
import functools

import jax
import jax.numpy as jnp
from jax import lax
from jax.experimental import pallas as pl
from jax.experimental.pallas import tpu as pltpu
from jax.experimental.pallas import tpu_sc as plsc

N_NODES = 10000
NPAD = 10240
N_EDGES = 320000
NCORES = 2
NSUB = 16
LANES = 16
HALF = 64
K = 80
EPS = N_EDGES // NSUB
NCHUNK = EPS // K
ROWS_PER_SUB = NPAD // NSUB


NSLOT = 4


def _sc_accumulate_body(with_cnt, xcat, src, dst, u, s_out, t_out, c_out,
                        s_sh, t_sh, c_sh, sidx, didx0, didx1, didx2, didx3,
                        ubuf, gbuf, tbuf, ones, zc, *sems):
  didxs = (didx0, didx1, didx2, didx3)
  idx_sems = sems[0:NSLOT]
  g_sems = sems[NSLOT:2 * NSLOT]
  scs_sems = sems[2 * NSLOT:3 * NSLOT]
  sct_sems = sems[3 * NSLOT:4 * NSLOT]
  c = lax.axis_index("c")
  s = lax.axis_index("s")

  zero16 = jnp.zeros((LANES,), jnp.float32)
  one16 = jnp.ones((LANES,), jnp.float32)
  for r in range(K):
    for j in range(HALF // LANES):
      gbuf[0, r, pl.ds(j * LANES, LANES)] = zero16
    zc[r, :] = zero16
    ones[r, :] = one16

  row0 = s * ROWS_PER_SUB
  for b in range(ROWS_PER_SUB // K):
    sl = pl.ds(row0 + b * K, K)
    pltpu.sync_copy(gbuf.at[0], s_sh.at[sl])
    pltpu.sync_copy(gbuf.at[0], t_sh.at[sl])
    if with_cnt:
      pltpu.sync_copy(zc, c_sh.at[sl])
  plsc.subcore_barrier()

  def issue_idx(g, b):
    base = s * EPS + g * K
    pltpu.async_copy(src.at[pl.ds(base, K)], sidx.at[b], idx_sems[b])
    pltpu.async_copy(dst.at[pl.ds(base, K)], didxs[b], idx_sems[b])
    pltpu.async_copy(u.at[pl.ds(base, K)], ubuf.at[b], idx_sems[b])

  def wait_idx(b):
    pltpu.make_async_copy(src.at[pl.ds(0, K)], sidx.at[b], idx_sems[b]).wait()
    pltpu.make_async_copy(dst.at[pl.ds(0, K)], didxs[b], idx_sems[b]).wait()
    pltpu.make_async_copy(u.at[pl.ds(0, K)], ubuf.at[b], idx_sems[b]).wait()

  def fixup(b):
    for j in range(K // LANES):
      sl = pl.ds(j * LANES, LANES)
      sidx[b, sl] = sidx[b, sl] * 2 + c

  def issue_gather(b):
    pltpu.async_copy(xcat.at[sidx.at[b]], gbuf.at[b], g_sems[b])

  def wait_gather(b):
    pltpu.make_async_copy(xcat.at[sidx.at[b]], gbuf.at[b], g_sems[b]).wait()

  def issue_scs(b):
    pltpu.async_copy(gbuf.at[b], s_sh.at[didxs[b]], scs_sems[b], add=True)
    if with_cnt:
      @pl.when(c == 0)
      def _():
        pltpu.async_copy(ones, c_sh.at[didxs[b]], scs_sems[b], add=True)

  def wait_scs(b):
    pltpu.make_async_copy(gbuf.at[b], s_sh.at[didxs[b]], scs_sems[b]).wait()
    if with_cnt:
      @pl.when(c == 0)
      def _():
        pltpu.make_async_copy(ones, c_sh.at[didxs[b]], scs_sems[b]).wait()

  def multiply(b):
    tb = b % 2
    def mul16(i16, carry):
      base_i = i16 * LANES
      uv = ubuf[b, pl.ds(base_i, LANES)]
      for di in range(LANES):
        i = base_i + di
        u16 = jnp.take_along_axis(uv, jnp.full((LANES,), di, jnp.int32),
                                  axis=0)
        for j in range(HALF // LANES):
          sl = pl.ds(j * LANES, LANES)
          tbuf[tb, i, sl] = gbuf[b, i, sl] * u16
      return carry
    lax.fori_loop(0, K // LANES, mul16, 0)

  def issue_sct(b):
    pltpu.async_copy(tbuf.at[b % 2], t_sh.at[didxs[b]], sct_sems[b], add=True)

  def wait_sct(b):
    pltpu.make_async_copy(tbuf.at[b % 2], t_sh.at[didxs[b]], sct_sems[b]).wait()

  issue_idx(0, 0)
  issue_idx(1, 1)
  wait_idx(0)
  fixup(0)
  issue_gather(0)

  def slot(g, b, drain=True, idx2=True, nxt=True):
    b1 = (b + 1) % NSLOT
    b2 = (b + 2) % NSLOT
    wait_gather(b)
    if nxt:
      wait_idx(b1)
      fixup(b1)
      issue_gather(b1)
    issue_scs(b)
    if drain:
      wait_sct(b2)
      wait_scs(b2)
    if idx2:
      issue_idx(g + 2, b2)
    multiply(b)
    issue_sct(b)

  def outer_body(oi, carry):
    for b in range(NSLOT):
      g = oi * NSLOT + b
      b1 = (b + 1) % NSLOT
      b2 = (b + 2) % NSLOT
      wait_gather(b)
      wait_idx(b1)
      fixup(b1)
      issue_gather(b1)
      issue_scs(b)

      @pl.when(g >= 2)
      def _():
        wait_sct(b2)
        wait_scs(b2)
      issue_idx(g + 2, b2)
      multiply(b)
      issue_sct(b)
    return carry

  assert (NCHUNK - 2) % NSLOT == 0
  lax.fori_loop(0, (NCHUNK - 2) // NSLOT, outer_body, 0)

  slot(NCHUNK - 2, (NCHUNK - 2) % NSLOT, idx2=False)
  slot(NCHUNK - 1, (NCHUNK - 1) % NSLOT, idx2=False, nxt=False)
  wait_sct((NCHUNK - 2) % NSLOT)
  wait_scs((NCHUNK - 2) % NSLOT)
  wait_sct((NCHUNK - 1) % NSLOT)
  wait_scs((NCHUNK - 1) % NSLOT)
  plsc.subcore_barrier()

  for b in range(ROWS_PER_SUB // K):
    r = row0 + b * K
    sl = pl.ds(r, K)
    csl = pl.ds(c * HALF, HALF)
    pltpu.sync_copy(s_sh.at[sl], s_out.at[pl.ds(r, K), csl])
    pltpu.sync_copy(t_sh.at[sl], t_out.at[pl.ds(r, K), csl])
    if with_cnt:
      @pl.when(c == 0)
      def _():
        pltpu.sync_copy(c_sh.at[sl], c_out.at[sl])


def _make_sc_accumulate(with_cnt):
  out_type = [
      jax.ShapeDtypeStruct((NPAD, NCORES * HALF), jnp.float32),
      jax.ShapeDtypeStruct((NPAD, NCORES * HALF), jnp.float32),
  ]
  if with_cnt:
    out_type.append(jax.ShapeDtypeStruct((NPAD, LANES), jnp.float32))
  scratch = [
      pltpu.VMEM_SHARED((NPAD, HALF), jnp.float32),
      pltpu.VMEM_SHARED((NPAD, HALF), jnp.float32),
      pltpu.VMEM_SHARED((NPAD, LANES), jnp.float32),
      pltpu.VMEM((NSLOT, K), jnp.int32),
      pltpu.VMEM((K,), jnp.int32),
      pltpu.VMEM((K,), jnp.int32),
      pltpu.VMEM((K,), jnp.int32),
      pltpu.VMEM((K,), jnp.int32),
      pltpu.VMEM((NSLOT, K), jnp.float32),
      pltpu.VMEM((NSLOT, K, HALF), jnp.float32),
      pltpu.VMEM((2, K, HALF), jnp.float32),
      pltpu.VMEM((K, LANES), jnp.float32),
      pltpu.VMEM((K, LANES), jnp.float32),
  ] + [pltpu.SemaphoreType.DMA] * (4 * NSLOT)
  mesh = plsc.VectorSubcoreMesh(
      core_axis_name="c", subcore_axis_name="s",
      num_cores=NCORES, num_subcores=NSUB)
  body = functools.partial(_sc_accumulate_body, with_cnt)
  if not with_cnt:
    def body(xcat, src, dst, u, s_out, t_out, *rest):
      return _sc_accumulate_body(False, xcat, src, dst, u, s_out, t_out,
                                 None, *rest)
  return pl.kernel(body, out_type=tuple(out_type), mesh=mesh,
                   scratch_types=tuple(scratch),
                   compiler_params=pltpu.CompilerParams(
                       use_tc_tiling_on_sc=False))


def _root_body(X_ref, Wr_ref, b_ref, o_ref):
  o_ref[...] = (jnp.dot(X_ref[...], Wr_ref[...],
                        preferred_element_type=jnp.float32) + b_ref[...])


def _make_root(d_in, d_out, br=1000):
  return pl.pallas_call(
      _root_body,
      grid=(N_NODES // br,),
      in_specs=[
          pl.BlockSpec((br, d_in), lambda i: (i, 0)),
          pl.BlockSpec((d_in, d_out), lambda i: (0, 0)),
          pl.BlockSpec((1, d_out), lambda i: (0, 0)),
      ],
      out_specs=pl.BlockSpec((br, d_out), lambda i: (i, 0)),
      out_shape=jax.ShapeDtypeStruct((N_NODES, d_out), jnp.float32),
  )


def _dense_body(last, S_ref, T_ref, R_ref, cnt_ref, W0_ref, W1_ref, o_ref):
  Sb = S_ref[...]
  Tb = T_ref[...]
  inv = 1.0 / jnp.clip(cnt_ref[...], 1.0, None)
  z = (jnp.dot(Sb - Tb, W0_ref[...], preferred_element_type=jnp.float32)
       + jnp.dot(Tb, W1_ref[...], preferred_element_type=jnp.float32))
  z = z * inv + R_ref[...]
  z = jnp.maximum(z, 0.0)
  if last:
    m = jnp.max(z, axis=1, keepdims=True)
    e = jnp.exp(z - m)
    z = z - m - jnp.log(jnp.sum(e, axis=1, keepdims=True))
  o_ref[...] = z


def _make_dense(d_in, d_out, last, br=1000):
  grid = (N_NODES // br,)
  return pl.pallas_call(
      functools.partial(_dense_body, last),
      grid=grid,
      in_specs=[
          pl.BlockSpec((br, d_in), lambda i: (i, 0)),
          pl.BlockSpec((br, d_in), lambda i: (i, 0)),
          pl.BlockSpec((br, d_out), lambda i: (i, 0)),
          pl.BlockSpec((br, 1), lambda i: (i, 0)),
          pl.BlockSpec((d_in, d_out), lambda i: (0, 0)),
          pl.BlockSpec((d_in, d_out), lambda i: (0, 0)),
      ],
      out_specs=pl.BlockSpec((br, d_out), lambda i: (i, 0)),
      out_shape=jax.ShapeDtypeStruct((N_NODES, d_out), jnp.float32),
  )


_make_sc_accumulate = functools.lru_cache(None)(_make_sc_accumulate)
_root1 = _make_root(128, 128)
_root2 = _make_root(128, 64)
_dense1 = _make_dense(128, 128, last=False)
_dense2 = _make_dense(128, 64, last=True)


def _split_cat(h):
  return h.reshape(2 * N_NODES, HALF)


def kernel(x, edge_index, edge_attr, W0_1, W1_1, Wr1, b1, W0_2, W1_2, Wr2, b2):
  src = edge_index[0]
  dst = edge_index[1]
  u = edge_attr[:, 0]

  R1 = _root1(x, Wr1, b1[None, :])
  S2, T2, C = _make_sc_accumulate(True)(_split_cat(x), src, dst, u)
  S = S2[:N_NODES]
  T = T2[:N_NODES]
  cnt = C[:N_NODES, 0:1]
  h = _dense1(S, T, R1, cnt, W0_1, W1_1)

  R2 = _root2(h, Wr2, b2[None, :])
  S2b, T2b = _make_sc_accumulate(False)(_split_cat(h), src, dst, u)
  Sb = S2b[:N_NODES]
  Tb = T2b[:N_NODES]
  return _dense2(Sb, Tb, R2, cnt, W0_2, W1_2)
